# Initial kernel scaffold; baseline (speedup 1.0000x reference)
import functools

import jax
import jax.numpy as jnp
from jax import lax
from jax.experimental import pallas as pl
from jax.experimental.pallas import tpu as pltpu
from jax.experimental.pallas import tpu_sc as plsc

N = 10000
E = 320000
D = 128
H = 8
C = 16
ED = 4

NC = 2
NS = 16
NW = NC * NS
EW = E // NW
K = 80
CH = EW // K
P = 10240
RPT = P // NS
AW = 144


def _prep_body(x_ref, w_ref, as_ref, ad_ref, xp_ref, asrc_ref, adst_ref):
    xp = jnp.dot(x_ref[...], w_ref[...], preferred_element_type=jnp.float32)
    xp_ref[...] = xp
    asrc_ref[...] = jnp.dot(xp, as_ref[...], preferred_element_type=jnp.float32)
    adst_ref[...] = jnp.dot(xp, ad_ref[...], preferred_element_type=jnp.float32)


def _edge_body(ea_ref, we_ref, ae_ref):
    ae_ref[...] = jnp.dot(ea_ref[...], we_ref[...],
                          preferred_element_type=jnp.float32)


def _sc_body(ei, asrc, adst, ae, xp, out,
             sidx, didx, gs, gd, aev, xpv, msg, acc, sem0, sem1, sem2):
    c = lax.axis_index("c")
    s = lax.axis_index("s")
    wid = s * NC + c

    zero16 = jnp.zeros((16,), jnp.float32)
    lanes = lax.broadcasted_iota(jnp.int32, (16,), 0)
    lmask = lanes < 8

    def _zrow(r, carry):
        for j in range(AW // 16):
            msg[r, pl.ds(j * 16, 16)] = zero16
        return carry
    lax.fori_loop(0, K, _zrow, 0)
    for j in range(RPT // K):
        pltpu.sync_copy(msg, acc.at[pl.ds(s * RPT + j * K, K), :])
    plsc.subcore_barrier()

    def _chunk(g, carry):
        base = wid * EW + g * K
        pltpu.sync_copy(ei.at[0, pl.ds(base, K)], sidx)
        pltpu.sync_copy(ei.at[1, pl.ds(base, K)], didx)
        pltpu.sync_copy(ae.at[pl.ds(base, K), :], aev)
        cp0 = pltpu.async_copy(asrc.at[sidx], gs, sem0)
        cp1 = pltpu.async_copy(adst.at[didx], gd, sem1)
        cp2 = pltpu.async_copy(xp.at[sidx], xpv, sem2)
        cp0.wait()
        cp1.wait()
        cp2.wait()

        def _edge(e, ecarry):
            a = gs[e, :] + gd[e, :] + aev[e, :]
            a = jnp.where(a > 0.0, a, 0.2 * a)
            w = jnp.exp(a)
            w = jnp.where(lmask, w, 0.0)
            msg[e, pl.ds(128, 16)] = w
            for h in range(H):
                wh = jnp.sum(jnp.where(lanes == h, w, 0.0))
                msg[e, pl.ds(h * 16, 16)] = wh * xpv[e, pl.ds(h * 16, 16)]
            return ecarry
        lax.fori_loop(0, K, _edge, 0)

        pltpu.sync_copy(msg, acc.at[didx], add=True)
        return carry
    lax.fori_loop(0, CH, _chunk, 0)

    plsc.subcore_barrier()
    pltpu.sync_copy(acc.at[pl.ds(s * RPT, RPT), :],
                    out.at[c, pl.ds(s * RPT, RPT), :])


def _bn_stats_body(p_ref, s8_ref, bias_ref, y_ref, sums_ref):
    p = p_ref[...]
    num = p[0, :, 0:128] + p[1, :, 0:128]
    den = p[0, :, 128:136] + p[1, :, 128:136]
    dexp = jnp.dot(den, s8_ref[...], preferred_element_type=jnp.float32)
    y = num / (dexp + 1e-16) + bias_ref[...]
    y_ref[...] = y
    sums_ref[0, 0, :] = jnp.sum(y, axis=0)
    sums_ref[0, 1, :] = jnp.sum(y * y, axis=0)


def _bn_apply_body(y_ref, x_ref, sums_ref, gamma_ref, beta_ref, out_ref):
    sums = sums_ref[...]
    mean = jnp.sum(sums[:, 0, :], axis=0) / float(N)
    var = jnp.sum(sums[:, 1, :], axis=0) / float(N) - mean * mean
    var = jnp.maximum(var, 0.0)
    o = (y_ref[...] - mean) * lax.rsqrt(var + 1e-5) * gamma_ref[...] \
        + beta_ref[...]
    o = jnp.where(o > 0.0, o, jnp.expm1(o))
    out_ref[...] = o + x_ref[...]


def kernel(x, edge_index, edge_attr, W, att_src, att_dst, W_edge, att_edge,
           bias, gamma, beta):
    blkdiag = jnp.kron(jnp.eye(H, dtype=jnp.float32),
                       jnp.ones((C, 1), dtype=jnp.float32))
    as8 = blkdiag * att_src.reshape(H * C)[:, None]
    ad8 = blkdiag * att_dst.reshape(H * C)[:, None]
    pad8 = jnp.zeros((D, 8), jnp.float32)
    as16 = jnp.concatenate([as8, pad8], axis=1)
    ad16 = jnp.concatenate([ad8, pad8], axis=1)
    we8 = jnp.sum(W_edge.reshape(ED, H, C) * att_edge.reshape(1, H, C),
                  axis=-1)
    we16 = jnp.concatenate([we8, jnp.zeros((ED, 8), jnp.float32)], axis=1)
    s8 = jnp.kron(jnp.eye(H, dtype=jnp.float32),
                  jnp.ones((1, C), dtype=jnp.float32))
    bias2 = bias.reshape(1, D)
    gamma2 = gamma.reshape(1, D)
    beta2 = beta.reshape(1, D)

    nb = 10
    bn_rows = N // nb
    xp, asrc, adst = pl.pallas_call(
        _prep_body,
        grid=(nb,),
        in_specs=[
            pl.BlockSpec((bn_rows, D), lambda i: (i, 0)),
            pl.BlockSpec((D, D), lambda i: (0, 0)),
            pl.BlockSpec((D, 16), lambda i: (0, 0)),
            pl.BlockSpec((D, 16), lambda i: (0, 0)),
        ],
        out_specs=[
            pl.BlockSpec((bn_rows, D), lambda i: (i, 0)),
            pl.BlockSpec((bn_rows, 16), lambda i: (i, 0)),
            pl.BlockSpec((bn_rows, 16), lambda i: (i, 0)),
        ],
        out_shape=[
            jax.ShapeDtypeStruct((N, D), jnp.float32),
            jax.ShapeDtypeStruct((N, 16), jnp.float32),
            jax.ShapeDtypeStruct((N, 16), jnp.float32),
        ],
    )(x, W, as16, ad16)

    eb = 40
    eb_rows = E // eb
    ae = pl.pallas_call(
        _edge_body,
        grid=(eb,),
        in_specs=[
            pl.BlockSpec((eb_rows, ED), lambda i: (i, 0)),
            pl.BlockSpec((ED, 16), lambda i: (0, 0)),
        ],
        out_specs=pl.BlockSpec((eb_rows, 16), lambda i: (i, 0)),
        out_shape=jax.ShapeDtypeStruct((E, 16), jnp.float32),
    )(edge_attr, we16)

    mesh = plsc.VectorSubcoreMesh(core_axis_name="c", subcore_axis_name="s")
    partial = pl.kernel(
        _sc_body,
        out_type=jax.ShapeDtypeStruct((NC, P, AW), jnp.float32),
        mesh=mesh,
        scratch_types=[
            pltpu.VMEM((K,), jnp.int32),
            pltpu.VMEM((K,), jnp.int32),
            pltpu.VMEM((K, 16), jnp.float32),
            pltpu.VMEM((K, 16), jnp.float32),
            pltpu.VMEM((K, 16), jnp.float32),
            pltpu.VMEM((K, D), jnp.float32),
            pltpu.VMEM((K, AW), jnp.float32),
            pltpu.VMEM_SHARED((P, AW), jnp.float32),
            pltpu.SemaphoreType.DMA,
            pltpu.SemaphoreType.DMA,
            pltpu.SemaphoreType.DMA,
        ],
    )(edge_index, asrc, adst, ae, xp)

    y, sums = pl.pallas_call(
        _bn_stats_body,
        grid=(nb,),
        in_specs=[
            pl.BlockSpec((NC, bn_rows, AW), lambda i: (0, i, 0)),
            pl.BlockSpec((8, D), lambda i: (0, 0)),
            pl.BlockSpec((1, D), lambda i: (0, 0)),
        ],
        out_specs=[
            pl.BlockSpec((bn_rows, D), lambda i: (i, 0)),
            pl.BlockSpec((1, 2, D), lambda i: (i, 0, 0)),
        ],
        out_shape=[
            jax.ShapeDtypeStruct((N, D), jnp.float32),
            jax.ShapeDtypeStruct((nb, 2, D), jnp.float32),
        ],
    )(partial, s8, bias2)

    out = pl.pallas_call(
        _bn_apply_body,
        grid=(nb,),
        in_specs=[
            pl.BlockSpec((bn_rows, D), lambda i: (i, 0)),
            pl.BlockSpec((bn_rows, D), lambda i: (i, 0)),
            pl.BlockSpec((nb, 2, D), lambda i: (0, 0, 0)),
            pl.BlockSpec((1, D), lambda i: (0, 0)),
            pl.BlockSpec((1, D), lambda i: (0, 0)),
        ],
        out_specs=pl.BlockSpec((bn_rows, D), lambda i: (i, 0)),
        out_shape=jax.ShapeDtypeStruct((N, D), jnp.float32),
    )(y, x, sums, gamma2, beta2)
    return out


# speedup vs baseline: 30.8007x; 30.8007x over previous
import functools

import jax
import jax.numpy as jnp
from jax import lax
from jax.experimental import pallas as pl
from jax.experimental.pallas import tpu as pltpu
from jax.experimental.pallas import tpu_sc as plsc

N = 10000
E = 320000
D = 128
H = 8
C = 16
ED = 4

NC = 2
NS = 16
NW = NC * NS
EW = E // NW
K = 80
CH = EW // K
P = 10240
RPT = P // NS
AW = 144


def _prep_body(x_ref, w_ref, as_ref, ad_ref, xp_ref, asrc_ref, adst_ref):
    xp = jnp.dot(x_ref[...], w_ref[...], preferred_element_type=jnp.float32)
    xp_ref[...] = xp
    asrc_ref[...] = jnp.dot(xp, as_ref[...], preferred_element_type=jnp.float32)
    adst_ref[...] = jnp.dot(xp, ad_ref[...], preferred_element_type=jnp.float32)


def _edge_body(ea_ref, we_ref, ae_ref):
    ae_ref[...] = jnp.dot(ea_ref[...], we_ref[...],
                          preferred_element_type=jnp.float32)


def _sc_body(src_h, dst_h, asrc, adst, ae, xp, out,
             sidx, didx, gs, gd, aev, xpv, msg, acc, sem0, sem1, sem2):
    c = lax.axis_index("c")
    s = lax.axis_index("s")
    wid = s * NC + c

    zero16 = jnp.zeros((16,), jnp.float32)
    lanes = lax.broadcasted_iota(jnp.int32, (16,), 0)
    lmask = lanes < 8
    hcol = [jnp.full((16,), 128 + h, jnp.int32) for h in range(H)]

    def _zrow(r, carry):
        for j in range(AW // 16):
            msg[r, pl.ds(j * 16, 16)] = zero16
        return carry
    lax.fori_loop(0, K, _zrow, 0)
    for j in range(RPT // K):
        pltpu.sync_copy(msg, acc.at[pl.ds(s * RPT + j * K, K), :])
    plsc.subcore_barrier()

    def _chunk(g, carry):
        base = wid * EW + g * K
        pltpu.sync_copy(src_h.at[pl.ds(base, K)], sidx)
        pltpu.sync_copy(dst_h.at[pl.ds(base, K)], didx)
        pltpu.sync_copy(ae.at[pl.ds(base, K), :], aev)
        cp0 = pltpu.async_copy(asrc.at[sidx], gs, sem0)
        cp1 = pltpu.async_copy(adst.at[didx], gd, sem1)
        cp2 = pltpu.async_copy(xp.at[sidx], xpv, sem2)
        cp0.wait()
        cp1.wait()
        cp2.wait()

        def _edge(e, ecarry):
            a = gs[e, :] + gd[e, :] + aev[e, :]
            a = jnp.where(a > 0.0, a, 0.2 * a)
            w = jnp.exp(a)
            w = jnp.where(lmask, w, 0.0)
            msg[e, pl.ds(128, 16)] = w
            erow = jnp.full((16,), e, jnp.int32)
            for h in range(H):
                wh = plsc.load_gather(msg, [erow, hcol[h]])
                msg[e, pl.ds(h * 16, 16)] = wh * xpv[e, pl.ds(h * 16, 16)]
            return ecarry
        lax.fori_loop(0, K, _edge, 0)

        pltpu.sync_copy(msg, acc.at[didx], add=True)
        return carry
    lax.fori_loop(0, CH, _chunk, 0)

    plsc.subcore_barrier()
    pltpu.sync_copy(acc.at[pl.ds(s * RPT, RPT), :],
                    out.at[c, pl.ds(s * RPT, RPT), :])


def _bn_stats_body(p_ref, s8_ref, bias_ref, y_ref, sums_ref):
    p = p_ref[...]
    num = p[0, :, 0:128] + p[1, :, 0:128]
    den = p[0, :, 128:136] + p[1, :, 128:136]
    dexp = jnp.dot(den, s8_ref[...], preferred_element_type=jnp.float32)
    y = num / (dexp + 1e-16) + bias_ref[...]
    y_ref[...] = y
    sums_ref[0, 0, :] = jnp.sum(y, axis=0)
    sums_ref[0, 1, :] = jnp.sum(y * y, axis=0)


def _bn_apply_body(y_ref, x_ref, sums_ref, gamma_ref, beta_ref, out_ref):
    sums = sums_ref[...]
    mean = jnp.sum(sums[:, 0, :], axis=0) / float(N)
    var = jnp.sum(sums[:, 1, :], axis=0) / float(N) - mean * mean
    var = jnp.maximum(var, 0.0)
    o = (y_ref[...] - mean) * lax.rsqrt(var + 1e-5) * gamma_ref[...] \
        + beta_ref[...]
    o = jnp.where(o > 0.0, o, jnp.exp(jnp.minimum(o, 0.0)) - 1.0)
    out_ref[...] = o + x_ref[...]


def kernel(x, edge_index, edge_attr, W, att_src, att_dst, W_edge, att_edge,
           bias, gamma, beta):
    blkdiag = jnp.kron(jnp.eye(H, dtype=jnp.float32),
                       jnp.ones((C, 1), dtype=jnp.float32))
    as8 = blkdiag * att_src.reshape(H * C)[:, None]
    ad8 = blkdiag * att_dst.reshape(H * C)[:, None]
    pad8 = jnp.zeros((D, 8), jnp.float32)
    as16 = jnp.concatenate([as8, pad8], axis=1)
    ad16 = jnp.concatenate([ad8, pad8], axis=1)
    we8 = jnp.sum(W_edge.reshape(ED, H, C) * att_edge.reshape(1, H, C),
                  axis=-1)
    we16 = jnp.concatenate([we8, jnp.zeros((ED, 8), jnp.float32)], axis=1)
    s8 = jnp.kron(jnp.eye(H, dtype=jnp.float32),
                  jnp.ones((1, C), dtype=jnp.float32))
    bias2 = bias.reshape(1, D)
    gamma2 = gamma.reshape(1, D)
    beta2 = beta.reshape(1, D)

    nb = 10
    bn_rows = N // nb
    xp, asrc, adst = pl.pallas_call(
        _prep_body,
        grid=(nb,),
        in_specs=[
            pl.BlockSpec((bn_rows, D), lambda i: (i, 0)),
            pl.BlockSpec((D, D), lambda i: (0, 0)),
            pl.BlockSpec((D, 16), lambda i: (0, 0)),
            pl.BlockSpec((D, 16), lambda i: (0, 0)),
        ],
        out_specs=[
            pl.BlockSpec((bn_rows, D), lambda i: (i, 0)),
            pl.BlockSpec((bn_rows, 16), lambda i: (i, 0)),
            pl.BlockSpec((bn_rows, 16), lambda i: (i, 0)),
        ],
        out_shape=[
            jax.ShapeDtypeStruct((N, D), jnp.float32),
            jax.ShapeDtypeStruct((N, 16), jnp.float32),
            jax.ShapeDtypeStruct((N, 16), jnp.float32),
        ],
    )(x, W, as16, ad16)

    eb = 40
    eb_rows = E // eb
    ae = pl.pallas_call(
        _edge_body,
        grid=(eb,),
        in_specs=[
            pl.BlockSpec((eb_rows, ED), lambda i: (i, 0)),
            pl.BlockSpec((ED, 16), lambda i: (0, 0)),
        ],
        out_specs=pl.BlockSpec((eb_rows, 16), lambda i: (i, 0)),
        out_shape=jax.ShapeDtypeStruct((E, 16), jnp.float32),
    )(edge_attr, we16)

    mesh = plsc.VectorSubcoreMesh(core_axis_name="c", subcore_axis_name="s")
    partial = pl.kernel(
        _sc_body,
        out_type=jax.ShapeDtypeStruct((NC, P, AW), jnp.float32),
        mesh=mesh,
        compiler_params=pltpu.CompilerParams(needs_layout_passes=False,
                                             use_tc_tiling_on_sc=False),
        scratch_types=[
            pltpu.VMEM((K,), jnp.int32),
            pltpu.VMEM((K,), jnp.int32),
            pltpu.VMEM((K, 16), jnp.float32),
            pltpu.VMEM((K, 16), jnp.float32),
            pltpu.VMEM((K, 16), jnp.float32),
            pltpu.VMEM((K, D), jnp.float32),
            pltpu.VMEM((K, AW), jnp.float32),
            pltpu.VMEM_SHARED((P, AW), jnp.float32),
            pltpu.SemaphoreType.DMA,
            pltpu.SemaphoreType.DMA,
            pltpu.SemaphoreType.DMA,
        ],
    )(edge_index[0], edge_index[1], asrc, adst, ae, xp)

    y, sums = pl.pallas_call(
        _bn_stats_body,
        grid=(nb,),
        in_specs=[
            pl.BlockSpec((NC, bn_rows, AW), lambda i: (0, i, 0)),
            pl.BlockSpec((8, D), lambda i: (0, 0)),
            pl.BlockSpec((1, D), lambda i: (0, 0)),
        ],
        out_specs=[
            pl.BlockSpec((bn_rows, D), lambda i: (i, 0)),
            pl.BlockSpec((1, 2, D), lambda i: (i, 0, 0)),
        ],
        out_shape=[
            jax.ShapeDtypeStruct((N, D), jnp.float32),
            jax.ShapeDtypeStruct((nb, 2, D), jnp.float32),
        ],
    )(partial, s8, bias2)

    out = pl.pallas_call(
        _bn_apply_body,
        grid=(nb,),
        in_specs=[
            pl.BlockSpec((bn_rows, D), lambda i: (i, 0)),
            pl.BlockSpec((bn_rows, D), lambda i: (i, 0)),
            pl.BlockSpec((nb, 2, D), lambda i: (0, 0, 0)),
            pl.BlockSpec((1, D), lambda i: (0, 0)),
            pl.BlockSpec((1, D), lambda i: (0, 0)),
        ],
        out_specs=pl.BlockSpec((bn_rows, D), lambda i: (i, 0)),
        out_shape=jax.ShapeDtypeStruct((N, D), jnp.float32),
    )(y, x, sums, gamma2, beta2)
    return out


# speedup vs baseline: 49.3093x; 1.6009x over previous
import functools

import jax
import jax.numpy as jnp
from jax import lax
from jax.experimental import pallas as pl
from jax.experimental.pallas import tpu as pltpu
from jax.experimental.pallas import tpu_sc as plsc

N = 10000
E = 320000
D = 128
H = 8
C = 16
ED = 4

NC = 2
NS = 16
NW = NC * NS
EW = E // NW
K = 40
CH = EW // K
P = 10240
RPT = P // NS
AW = 144


def _prep_body(x_ref, w_ref, as_ref, ad_ref, xp_ref, asrc_ref, adst_ref):
    xp = jnp.dot(x_ref[...], w_ref[...], preferred_element_type=jnp.float32)
    xp_ref[:, 0:D] = xp
    xp_ref[:, D:AW] = jnp.zeros((xp.shape[0], AW - D), jnp.float32)
    asrc_ref[...] = jnp.dot(xp, as_ref[...], preferred_element_type=jnp.float32)
    adst_ref[...] = jnp.dot(xp, ad_ref[...], preferred_element_type=jnp.float32)


def _edge_body(ea_ref, we_ref, ae_ref):
    ae_ref[...] = jnp.dot(ea_ref[...], we_ref[...],
                          preferred_element_type=jnp.float32)


NB = 4

_GDN = lax.GatherDimensionNumbers(offset_dims=(), collapsed_slice_dims=(0,),
                                  start_index_map=(0,))


def _bcast(v, h):
    idx = jnp.full((16, 1), h, jnp.int32)
    return lax.gather(v, idx, _GDN, (1,),
                      mode=lax.GatherScatterMode.PROMISE_IN_BOUNDS)


def _sc_body(src_h, dst_h, asrc, adst, ae, xp, out,
             sidx3, didx3, gs3, gd3, aev3, msg3, acc,
             sg0, sg1, sg2, sg3, ss0, ss1, ss2, ss3):
    c = lax.axis_index("c")
    s = lax.axis_index("s")
    wid = s * NC + c
    ebase = wid * EW
    semg = [sg0, sg1, sg2, sg3]
    sems = [ss0, ss1, ss2, ss3]

    zero16 = jnp.zeros((16,), jnp.float32)
    lanes = lax.broadcasted_iota(jnp.int32, (16,), 0)
    lmask = lanes < 8

    def _zrow(r, carry):
        for j in range(AW // 16):
            msg3[0, r, pl.ds(j * 16, 16)] = zero16
        return carry
    lax.fori_loop(0, K, _zrow, 0)
    for j in range(RPT // K):
        pltpu.sync_copy(msg3.at[0], acc.at[pl.ds(s * RPT + j * K, K), :])
    plsc.subcore_barrier()

    def _load_idx(g, b):
        base = ebase + g * K
        pltpu.sync_copy(src_h.at[pl.ds(base, K)], sidx3.at[b])
        pltpu.sync_copy(dst_h.at[pl.ds(base, K)], didx3.at[b])

    def _gather_pairs(g, b):
        base = ebase + g * K
        return [
            (asrc.at[sidx3.at[b]], gs3.at[b]),
            (adst.at[didx3.at[b]], gd3.at[b]),
            (xp.at[sidx3.at[b]], msg3.at[b]),
            (ae.at[pl.ds(base, K), :], aev3.at[b]),
        ]

    def _issue_g(g, b):
        for src, dst in _gather_pairs(g, b):
            pltpu.async_copy(src, dst, semg[b])

    def _wait_g(g, b):
        for src, dst in _gather_pairs(g, b):
            pltpu.make_async_copy(src, dst, semg[b]).wait()

    def _issue_s(b):
        pltpu.async_copy(msg3.at[b], acc.at[didx3.at[b]], sems[b], add=True)

    def _wait_s(b):
        pltpu.make_async_copy(msg3.at[b], acc.at[didx3.at[b]], sems[b]).wait()

    def _compute(b):
        def _edge(e, ecarry):
            a = gs3[b, e, :] + gd3[b, e, :] + aev3[b, e, :]
            a = jnp.where(a > 0.0, a, 0.2 * a)
            w = jnp.exp(a)
            w = jnp.where(lmask, w, 0.0)
            msg3[b, e, pl.ds(128, 16)] = w
            for h in range(H):
                msg3[b, e, pl.ds(h * 16, 16)] = \
                    _bcast(w, h) * msg3[b, e, pl.ds(h * 16, 16)]
            return ecarry
        lax.fori_loop(0, K, _edge, 0)

    _load_idx(0, 0)
    _issue_g(0, 0)
    _load_idx(1, 1)
    _issue_g(1, 1)

    NSLOT = (CH + NB - 1) // NB * NB
    def _slot_group(t, carry):
        for b in range(NB):
            g = t * NB + b

            @pl.when(jnp.logical_and(g >= 2, g - 2 < CH))
            def _():
                _wait_s((b + 2) % NB)

            @pl.when(g + 2 < CH)
            def _():
                _load_idx(g + 2, (b + 2) % NB)
                _issue_g(g + 2, (b + 2) % NB)

            @pl.when(g < CH)
            def _():
                _wait_g(g, b)
                _compute(b)
                _issue_s(b)
        return carry
    lax.fori_loop(0, NSLOT // NB, _slot_group, 0)

    plsc.subcore_barrier()
    pltpu.sync_copy(acc.at[pl.ds(s * RPT, RPT), :],
                    out.at[c, pl.ds(s * RPT, RPT), :])


def _bn_stats_body(p_ref, s8_ref, bias_ref, y_ref, sums_ref):
    p = p_ref[...]
    num = p[0, :, 0:128] + p[1, :, 0:128]
    den = p[0, :, 128:136] + p[1, :, 128:136]
    dexp = jnp.dot(den, s8_ref[...], preferred_element_type=jnp.float32)
    y = num / (dexp + 1e-16) + bias_ref[...]
    y_ref[...] = y
    sums_ref[0, 0, :] = jnp.sum(y, axis=0)
    sums_ref[0, 1, :] = jnp.sum(y * y, axis=0)


def _bn_apply_body(y_ref, x_ref, sums_ref, gamma_ref, beta_ref, out_ref):
    sums = sums_ref[...]
    mean = jnp.sum(sums[:, 0, :], axis=0) / float(N)
    var = jnp.sum(sums[:, 1, :], axis=0) / float(N) - mean * mean
    var = jnp.maximum(var, 0.0)
    o = (y_ref[...] - mean) * lax.rsqrt(var + 1e-5) * gamma_ref[...] \
        + beta_ref[...]
    o = jnp.where(o > 0.0, o, jnp.exp(jnp.minimum(o, 0.0)) - 1.0)
    out_ref[...] = o + x_ref[...]


def kernel(x, edge_index, edge_attr, W, att_src, att_dst, W_edge, att_edge,
           bias, gamma, beta):
    blkdiag = jnp.kron(jnp.eye(H, dtype=jnp.float32),
                       jnp.ones((C, 1), dtype=jnp.float32))
    as8 = blkdiag * att_src.reshape(H * C)[:, None]
    ad8 = blkdiag * att_dst.reshape(H * C)[:, None]
    pad8 = jnp.zeros((D, 8), jnp.float32)
    as16 = jnp.concatenate([as8, pad8], axis=1)
    ad16 = jnp.concatenate([ad8, pad8], axis=1)
    we8 = jnp.sum(W_edge.reshape(ED, H, C) * att_edge.reshape(1, H, C),
                  axis=-1)
    we16 = jnp.concatenate([we8, jnp.zeros((ED, 8), jnp.float32)], axis=1)
    s8 = jnp.kron(jnp.eye(H, dtype=jnp.float32),
                  jnp.ones((1, C), dtype=jnp.float32))
    bias2 = bias.reshape(1, D)
    gamma2 = gamma.reshape(1, D)
    beta2 = beta.reshape(1, D)

    nb = 10
    bn_rows = N // nb
    xp, asrc, adst = pl.pallas_call(
        _prep_body,
        grid=(nb,),
        in_specs=[
            pl.BlockSpec((bn_rows, D), lambda i: (i, 0)),
            pl.BlockSpec((D, D), lambda i: (0, 0)),
            pl.BlockSpec((D, 16), lambda i: (0, 0)),
            pl.BlockSpec((D, 16), lambda i: (0, 0)),
        ],
        out_specs=[
            pl.BlockSpec((bn_rows, AW), lambda i: (i, 0)),
            pl.BlockSpec((bn_rows, 16), lambda i: (i, 0)),
            pl.BlockSpec((bn_rows, 16), lambda i: (i, 0)),
        ],
        out_shape=[
            jax.ShapeDtypeStruct((N, AW), jnp.float32),
            jax.ShapeDtypeStruct((N, 16), jnp.float32),
            jax.ShapeDtypeStruct((N, 16), jnp.float32),
        ],
    )(x, W, as16, ad16)

    eb = 40
    eb_rows = E // eb
    ae = pl.pallas_call(
        _edge_body,
        grid=(eb,),
        in_specs=[
            pl.BlockSpec((eb_rows, ED), lambda i: (i, 0)),
            pl.BlockSpec((ED, 16), lambda i: (0, 0)),
        ],
        out_specs=pl.BlockSpec((eb_rows, 16), lambda i: (i, 0)),
        out_shape=jax.ShapeDtypeStruct((E, 16), jnp.float32),
    )(edge_attr, we16)

    mesh = plsc.VectorSubcoreMesh(core_axis_name="c", subcore_axis_name="s")
    partial = pl.kernel(
        _sc_body,
        out_type=jax.ShapeDtypeStruct((NC, P, AW), jnp.float32),
        mesh=mesh,
        compiler_params=pltpu.CompilerParams(needs_layout_passes=False,
                                             use_tc_tiling_on_sc=False),
        scratch_types=[
            pltpu.VMEM((NB, K), jnp.int32),
            pltpu.VMEM((NB, K), jnp.int32),
            pltpu.VMEM((NB, K, 16), jnp.float32),
            pltpu.VMEM((NB, K, 16), jnp.float32),
            pltpu.VMEM((NB, K, 16), jnp.float32),
            pltpu.VMEM((NB, K, AW), jnp.float32),
            pltpu.VMEM_SHARED((P, AW), jnp.float32),
            pltpu.SemaphoreType.DMA,
            pltpu.SemaphoreType.DMA,
            pltpu.SemaphoreType.DMA,
            pltpu.SemaphoreType.DMA,
            pltpu.SemaphoreType.DMA,
            pltpu.SemaphoreType.DMA,
            pltpu.SemaphoreType.DMA,
            pltpu.SemaphoreType.DMA,
        ],
    )(edge_index[0], edge_index[1], asrc, adst, ae, xp)

    y, sums = pl.pallas_call(
        _bn_stats_body,
        grid=(nb,),
        in_specs=[
            pl.BlockSpec((NC, bn_rows, AW), lambda i: (0, i, 0)),
            pl.BlockSpec((8, D), lambda i: (0, 0)),
            pl.BlockSpec((1, D), lambda i: (0, 0)),
        ],
        out_specs=[
            pl.BlockSpec((bn_rows, D), lambda i: (i, 0)),
            pl.BlockSpec((1, 2, D), lambda i: (i, 0, 0)),
        ],
        out_shape=[
            jax.ShapeDtypeStruct((N, D), jnp.float32),
            jax.ShapeDtypeStruct((nb, 2, D), jnp.float32),
        ],
    )(partial, s8, bias2)

    out = pl.pallas_call(
        _bn_apply_body,
        grid=(nb,),
        in_specs=[
            pl.BlockSpec((bn_rows, D), lambda i: (i, 0)),
            pl.BlockSpec((bn_rows, D), lambda i: (i, 0)),
            pl.BlockSpec((nb, 2, D), lambda i: (0, 0, 0)),
            pl.BlockSpec((1, D), lambda i: (0, 0)),
            pl.BlockSpec((1, D), lambda i: (0, 0)),
        ],
        out_specs=pl.BlockSpec((bn_rows, D), lambda i: (i, 0)),
        out_shape=jax.ShapeDtypeStruct((N, D), jnp.float32),
    )(y, x, sums, gamma2, beta2)
    return out


# speedup vs baseline: 64.3073x; 1.3042x over previous
import functools

import jax
import jax.numpy as jnp
from jax import lax
from jax.experimental import pallas as pl
from jax.experimental.pallas import tpu as pltpu
from jax.experimental.pallas import tpu_sc as plsc

N = 10000
E = 320000
D = 128
H = 8
C = 16
ED = 4

NC = 2
NS = 16
NW = NC * NS
EW = E // NW
K = 40
CH = EW // K
P = 10240
RPT = P // NS
AW = 144


def _prep_body(x_ref, w_ref, as_ref, ad_ref, xp_ref, asrc_ref, adst_ref):
    xp = jnp.dot(x_ref[...], w_ref[...], preferred_element_type=jnp.float32)
    xp_ref[:, 0:D] = xp
    xp_ref[:, D:AW] = jnp.zeros((xp.shape[0], AW - D), jnp.float32)
    asrc_ref[...] = jnp.dot(xp, as_ref[...], preferred_element_type=jnp.float32)
    adst_ref[...] = jnp.dot(xp, ad_ref[...], preferred_element_type=jnp.float32)


def _edge_body(ea_ref, we_ref, ae_ref):
    ae_ref[...] = jnp.dot(ea_ref[...], we_ref[...],
                          preferred_element_type=jnp.float32)


NB = 4

_GDN = lax.GatherDimensionNumbers(offset_dims=(), collapsed_slice_dims=(0,),
                                  start_index_map=(0,))


def _bcast(v, h):
    idx = jnp.full((16, 1), h, jnp.int32)
    return lax.gather(v, idx, _GDN, (1,),
                      mode=lax.GatherScatterMode.PROMISE_IN_BOUNDS)


NI = 6


def _sc_body(src_h, dst_h, asrc, adst, ae, xp, out,
             sidx3, didx3, gs3, gd3, aev3, msg3, acc,
             sg0, sg1, sg2, sg3, ss0, ss1, ss2, ss3,
             sl0, sl1, sl2, sl3, sl4, sl5):
    c = lax.axis_index("c")
    s = lax.axis_index("s")
    wid = s * NC + c
    ebase = wid * EW
    semg = [sg0, sg1, sg2, sg3]
    sems = [ss0, ss1, ss2, ss3]
    seml = [sl0, sl1, sl2, sl3, sl4, sl5]

    zero16 = jnp.zeros((16,), jnp.float32)
    lanes = lax.broadcasted_iota(jnp.int32, (16,), 0)
    lmask = lanes < 8

    def _zrow(r, carry):
        for j in range(AW // 16):
            msg3[0, r, pl.ds(j * 16, 16)] = zero16
        return carry
    lax.fori_loop(0, K, _zrow, 0)
    for j in range(RPT // K):
        pltpu.sync_copy(msg3.at[0], acc.at[pl.ds(s * RPT + j * K, K), :])
    plsc.subcore_barrier()

    def _idx_pairs(g, bi):
        base = ebase + g * K
        return [
            (src_h.at[pl.ds(base, K)], sidx3.at[bi]),
            (dst_h.at[pl.ds(base, K)], didx3.at[bi]),
        ]

    def _issue_l(g, bi):
        for src, dst in _idx_pairs(g, bi):
            pltpu.async_copy(src, dst, seml[bi])

    def _wait_l(g, bi):
        for src, dst in _idx_pairs(g, bi):
            pltpu.make_async_copy(src, dst, seml[bi]).wait()

    def _gather_pairs(g, b, bi):
        base = ebase + g * K
        return [
            (asrc.at[sidx3.at[bi]], gs3.at[b]),
            (adst.at[didx3.at[bi]], gd3.at[b]),
            (xp.at[sidx3.at[bi]], msg3.at[b]),
            (ae.at[pl.ds(base, K), :], aev3.at[b]),
        ]

    def _issue_g(g, b, bi):
        for src, dst in _gather_pairs(g, b, bi):
            pltpu.async_copy(src, dst, semg[b])

    def _wait_g(g, b, bi):
        for src, dst in _gather_pairs(g, b, bi):
            pltpu.make_async_copy(src, dst, semg[b]).wait()

    def _issue_s(b, bi):
        pltpu.async_copy(msg3.at[b], acc.at[didx3.at[bi]], sems[b], add=True)

    def _wait_s(b, bi):
        pltpu.make_async_copy(msg3.at[b], acc.at[didx3.at[bi]], sems[b]).wait()

    def _compute(b):
        def _edge(i, ecarry):
            for u in range(2):
                e = 2 * i + u
                a = gs3[b, e, :] + gd3[b, e, :] + aev3[b, e, :]
                a = jnp.where(a > 0.0, a, 0.2 * a)
                w = jnp.exp(a)
                w = jnp.where(lmask, w, 0.0)
                msg3[b, e, pl.ds(128, 16)] = w
                for h in range(H):
                    msg3[b, e, pl.ds(h * 16, 16)] = \
                        _bcast(w, h) * msg3[b, e, pl.ds(h * 16, 16)]
            return ecarry
        lax.fori_loop(0, K // 2, _edge, 0)

    for g0 in range(4):
        _issue_l(g0, g0)
    _wait_l(0, 0)
    _wait_l(1, 1)
    _issue_g(0, 0, 0)
    _issue_g(1, 1, 1)

    NGRP = (CH + 2 + 11) // 12
    def _slot_group(t, carry):
        for j in range(12):
            g = t * 12 + j
            b = j % NB
            bi = j % NI

            @pl.when(jnp.logical_and(g >= 2, g - 2 < CH))
            def _():
                _wait_s((j + 2) % NB, (j + 4) % NI)

            @pl.when(g + 4 < CH)
            def _():
                _issue_l(g + 4, (j + 4) % NI)

            @pl.when(g + 2 < CH)
            def _():
                _wait_l(g + 2, (j + 2) % NI)
                _issue_g(g + 2, (j + 2) % NB, (j + 2) % NI)

            @pl.when(g < CH)
            def _():
                _wait_g(g, b, bi)
                _compute(b)
                _issue_s(b, bi)
        return carry
    lax.fori_loop(0, NGRP, _slot_group, 0)

    plsc.subcore_barrier()
    pltpu.sync_copy(acc.at[pl.ds(s * RPT, RPT), :],
                    out.at[c, pl.ds(s * RPT, RPT), :])


def _bn_stats_body(p_ref, s8_ref, bias_ref, y_ref, sums_ref):
    p = p_ref[...]
    num = p[0, :, 0:128] + p[1, :, 0:128]
    den = p[0, :, 128:136] + p[1, :, 128:136]
    dexp = jnp.dot(den, s8_ref[...], preferred_element_type=jnp.float32)
    y = num / (dexp + 1e-16) + bias_ref[...]
    y_ref[...] = y
    sums_ref[0, 0, :] = jnp.sum(y, axis=0)
    sums_ref[0, 1, :] = jnp.sum(y * y, axis=0)


def _bn_apply_body(y_ref, x_ref, sums_ref, gamma_ref, beta_ref, out_ref):
    sums = sums_ref[...]
    mean = jnp.sum(sums[:, 0, :], axis=0) / float(N)
    var = jnp.sum(sums[:, 1, :], axis=0) / float(N) - mean * mean
    var = jnp.maximum(var, 0.0)
    o = (y_ref[...] - mean) * lax.rsqrt(var + 1e-5) * gamma_ref[...] \
        + beta_ref[...]
    o = jnp.where(o > 0.0, o, jnp.exp(jnp.minimum(o, 0.0)) - 1.0)
    out_ref[...] = o + x_ref[...]


def kernel(x, edge_index, edge_attr, W, att_src, att_dst, W_edge, att_edge,
           bias, gamma, beta):
    blkdiag = jnp.kron(jnp.eye(H, dtype=jnp.float32),
                       jnp.ones((C, 1), dtype=jnp.float32))
    as8 = blkdiag * att_src.reshape(H * C)[:, None]
    ad8 = blkdiag * att_dst.reshape(H * C)[:, None]
    pad8 = jnp.zeros((D, 8), jnp.float32)
    as16 = jnp.concatenate([as8, pad8], axis=1)
    ad16 = jnp.concatenate([ad8, pad8], axis=1)
    we8 = jnp.sum(W_edge.reshape(ED, H, C) * att_edge.reshape(1, H, C),
                  axis=-1)
    we16 = jnp.concatenate([we8, jnp.zeros((ED, 8), jnp.float32)], axis=1)
    s8 = jnp.kron(jnp.eye(H, dtype=jnp.float32),
                  jnp.ones((1, C), dtype=jnp.float32))
    bias2 = bias.reshape(1, D)
    gamma2 = gamma.reshape(1, D)
    beta2 = beta.reshape(1, D)

    nb = 10
    bn_rows = N // nb
    xp, asrc, adst = pl.pallas_call(
        _prep_body,
        grid=(nb,),
        in_specs=[
            pl.BlockSpec((bn_rows, D), lambda i: (i, 0)),
            pl.BlockSpec((D, D), lambda i: (0, 0)),
            pl.BlockSpec((D, 16), lambda i: (0, 0)),
            pl.BlockSpec((D, 16), lambda i: (0, 0)),
        ],
        out_specs=[
            pl.BlockSpec((bn_rows, AW), lambda i: (i, 0)),
            pl.BlockSpec((bn_rows, 16), lambda i: (i, 0)),
            pl.BlockSpec((bn_rows, 16), lambda i: (i, 0)),
        ],
        out_shape=[
            jax.ShapeDtypeStruct((N, AW), jnp.float32),
            jax.ShapeDtypeStruct((N, 16), jnp.float32),
            jax.ShapeDtypeStruct((N, 16), jnp.float32),
        ],
    )(x, W, as16, ad16)

    eb = 40
    eb_rows = E // eb
    ae = pl.pallas_call(
        _edge_body,
        grid=(eb,),
        in_specs=[
            pl.BlockSpec((eb_rows, ED), lambda i: (i, 0)),
            pl.BlockSpec((ED, 16), lambda i: (0, 0)),
        ],
        out_specs=pl.BlockSpec((eb_rows, 16), lambda i: (i, 0)),
        out_shape=jax.ShapeDtypeStruct((E, 16), jnp.float32),
    )(edge_attr, we16)

    mesh = plsc.VectorSubcoreMesh(core_axis_name="c", subcore_axis_name="s")
    partial = pl.kernel(
        _sc_body,
        out_type=jax.ShapeDtypeStruct((NC, P, AW), jnp.float32),
        mesh=mesh,
        compiler_params=pltpu.CompilerParams(needs_layout_passes=False,
                                             use_tc_tiling_on_sc=False),
        scratch_types=(
            [
                pltpu.VMEM((NI, K), jnp.int32),
                pltpu.VMEM((NI, K), jnp.int32),
                pltpu.VMEM((NB, K, 16), jnp.float32),
                pltpu.VMEM((NB, K, 16), jnp.float32),
                pltpu.VMEM((NB, K, 16), jnp.float32),
                pltpu.VMEM((NB, K, AW), jnp.float32),
                pltpu.VMEM_SHARED((P, AW), jnp.float32),
            ]
            + [pltpu.SemaphoreType.DMA] * (2 * NB + NI)
        ),
    )(edge_index[0], edge_index[1], asrc, adst, ae, xp)

    y, sums = pl.pallas_call(
        _bn_stats_body,
        grid=(nb,),
        in_specs=[
            pl.BlockSpec((NC, bn_rows, AW), lambda i: (0, i, 0)),
            pl.BlockSpec((8, D), lambda i: (0, 0)),
            pl.BlockSpec((1, D), lambda i: (0, 0)),
        ],
        out_specs=[
            pl.BlockSpec((bn_rows, D), lambda i: (i, 0)),
            pl.BlockSpec((1, 2, D), lambda i: (i, 0, 0)),
        ],
        out_shape=[
            jax.ShapeDtypeStruct((N, D), jnp.float32),
            jax.ShapeDtypeStruct((nb, 2, D), jnp.float32),
        ],
    )(partial, s8, bias2)

    out = pl.pallas_call(
        _bn_apply_body,
        grid=(nb,),
        in_specs=[
            pl.BlockSpec((bn_rows, D), lambda i: (i, 0)),
            pl.BlockSpec((bn_rows, D), lambda i: (i, 0)),
            pl.BlockSpec((nb, 2, D), lambda i: (0, 0, 0)),
            pl.BlockSpec((1, D), lambda i: (0, 0)),
            pl.BlockSpec((1, D), lambda i: (0, 0)),
        ],
        out_specs=pl.BlockSpec((bn_rows, D), lambda i: (i, 0)),
        out_shape=jax.ShapeDtypeStruct((N, D), jnp.float32),
    )(y, x, sums, gamma2, beta2)
    return out


# speedup vs baseline: 75.1181x; 1.1681x over previous
import functools

import jax
import jax.numpy as jnp
from jax import lax
from jax.experimental import pallas as pl
from jax.experimental.pallas import tpu as pltpu
from jax.experimental.pallas import tpu_sc as plsc

N = 10000
E = 320000
D = 128
H = 8
C = 16
ED = 4

NC = 2
NS = 16
NW = NC * NS
EW = E // NW
K = 40
CH = EW // K
P = 10240
RPT = P // NS
AW = 144


def _prep_body(x_ref, w_ref, as_ref, ad_ref, xp_ref, asrc_ref, adst_ref):
    xp = jnp.dot(x_ref[...], w_ref[...], preferred_element_type=jnp.float32)
    xp_ref[...] = xp
    asrc_ref[...] = jnp.dot(xp, as_ref[...], preferred_element_type=jnp.float32)
    adst_ref[...] = jnp.dot(xp, ad_ref[...], preferred_element_type=jnp.float32)


def _edge_body(ea_ref, we_ref, ae_ref):
    ae_ref[...] = jnp.dot(ea_ref[...], we_ref[...],
                          preferred_element_type=jnp.float32)


NB = 4

_GDN = lax.GatherDimensionNumbers(offset_dims=(), collapsed_slice_dims=(0,),
                                  start_index_map=(0,))


def _bcast(v, h):
    idx = jnp.full((16, 1), h, jnp.int32)
    return lax.gather(v, idx, _GDN, (1,),
                      mode=lax.GatherScatterMode.PROMISE_IN_BOUNDS)


NI = 6


def _sc_body(src_h, dst_h, asrc, adst, ae, xp, out_m, out_d,
             sidx3, didx3, gs3, gd3, aev3, msg3, wden3, acc_m, acc_d,
             sg0, sg1, sg2, sg3, ss0, ss1, ss2, ss3,
             sl0, sl1, sl2, sl3, sl4, sl5):
    c = lax.axis_index("c")
    s = lax.axis_index("s")
    wid = s * NC + c
    ebase = wid * EW
    semg = [sg0, sg1, sg2, sg3]
    sems = [ss0, ss1, ss2, ss3]
    seml = [sl0, sl1, sl2, sl3, sl4, sl5]

    zero16 = jnp.zeros((16,), jnp.float32)
    lanes = lax.broadcasted_iota(jnp.int32, (16,), 0)
    lmask = lanes < 8

    def _zrow(r, carry):
        for j in range(D // 16):
            msg3[0, r, pl.ds(j * 16, 16)] = zero16
        wden3[0, r, :] = zero16
        return carry
    lax.fori_loop(0, K, _zrow, 0)
    for j in range(RPT // K):
        pltpu.sync_copy(msg3.at[0], acc_m.at[pl.ds(s * RPT + j * K, K), :])
        pltpu.sync_copy(wden3.at[0], acc_d.at[pl.ds(s * RPT + j * K, K), :])
    plsc.subcore_barrier()

    def _idx_pairs(g, bi):
        base = ebase + g * K
        return [
            (src_h.at[pl.ds(base, K)], sidx3.at[bi]),
            (dst_h.at[pl.ds(base, K)], didx3.at[bi]),
        ]

    def _issue_l(g, bi):
        for src, dst in _idx_pairs(g, bi):
            pltpu.async_copy(src, dst, seml[bi])

    def _wait_l(g, bi):
        for src, dst in _idx_pairs(g, bi):
            pltpu.make_async_copy(src, dst, seml[bi]).wait()

    def _gather_pairs(g, b, bi):
        base = ebase + g * K
        return [
            (asrc.at[sidx3.at[bi]], gs3.at[b]),
            (adst.at[didx3.at[bi]], gd3.at[b]),
            (xp.at[sidx3.at[bi]], msg3.at[b]),
            (ae.at[pl.ds(base // 8, K // 8), :], aev3.at[b]),
        ]

    def _issue_g(g, b, bi):
        for src, dst in _gather_pairs(g, b, bi):
            pltpu.async_copy(src, dst, semg[b])

    def _wait_g(g, b, bi):
        for src, dst in _gather_pairs(g, b, bi):
            pltpu.make_async_copy(src, dst, semg[b]).wait()

    def _scatter_pairs(b, bi):
        return [
            (msg3.at[b], acc_m.at[didx3.at[bi]]),
            (wden3.at[b], acc_d.at[didx3.at[bi]]),
        ]

    def _issue_s(b, bi):
        for src, dst in _scatter_pairs(b, bi):
            pltpu.async_copy(src, dst, sems[b], add=True)

    def _wait_s(b, bi):
        for src, dst in _scatter_pairs(b, bi):
            pltpu.make_async_copy(src, dst, sems[b]).wait()

    def _compute(b):
        def _edge(i, ecarry):
            for u in range(2):
                e = 2 * i + u
                r = e >> 3
                col = (e & 7) * 16
                a = gs3[b, e, :] + gd3[b, e, :] + aev3[b, r, pl.ds(col, 16)]
                a = jnp.where(a > 0.0, a, 0.2 * a)
                w = jnp.exp(a)
                w = jnp.where(lmask, w, 0.0)
                wden3[b, e, :] = w
                for h in range(H):
                    msg3[b, e, pl.ds(h * 16, 16)] = \
                        _bcast(w, h) * msg3[b, e, pl.ds(h * 16, 16)]
            return ecarry
        lax.fori_loop(0, K // 2, _edge, 0)

    for g0 in range(4):
        _issue_l(g0, g0)
    _wait_l(0, 0)
    _wait_l(1, 1)
    _issue_g(0, 0, 0)
    _issue_g(1, 1, 1)

    NGRP = (CH + 2 + 11) // 12
    def _slot_group(t, carry):
        for j in range(12):
            g = t * 12 + j
            b = j % NB
            bi = j % NI

            @pl.when(jnp.logical_and(g >= 2, g - 2 < CH))
            def _():
                _wait_s((j + 2) % NB, (j + 4) % NI)

            @pl.when(g + 4 < CH)
            def _():
                _issue_l(g + 4, (j + 4) % NI)

            @pl.when(g + 2 < CH)
            def _():
                _wait_l(g + 2, (j + 2) % NI)
                _issue_g(g + 2, (j + 2) % NB, (j + 2) % NI)

            @pl.when(g < CH)
            def _():
                _wait_g(g, b, bi)
                _compute(b)
                _issue_s(b, bi)
        return carry
    lax.fori_loop(0, NGRP, _slot_group, 0)

    plsc.subcore_barrier()
    pltpu.sync_copy(acc_m.at[pl.ds(s * RPT, RPT), :],
                    out_m.at[c, pl.ds(s * RPT, RPT), :])
    pltpu.sync_copy(acc_d.at[pl.ds(s * RPT, RPT), :],
                    out_d.at[c, pl.ds(s * RPT, RPT), :])


def _bn_stats_body(pm_ref, pd_ref, s8_ref, bias_ref, y_ref, sums_ref):
    pm = pm_ref[...]
    pd = pd_ref[...]
    num = pm[0] + pm[1]
    den = pd[0, :, 0:8] + pd[1, :, 0:8]
    dexp = jnp.dot(den, s8_ref[...], preferred_element_type=jnp.float32)
    y = num / (dexp + 1e-16) + bias_ref[...]
    y_ref[...] = y
    sums_ref[0, 0, :] = jnp.sum(y, axis=0)
    sums_ref[0, 1, :] = jnp.sum(y * y, axis=0)


def _bn_apply_body(y_ref, x_ref, sums_ref, gamma_ref, beta_ref, out_ref):
    sums = sums_ref[...]
    mean = jnp.sum(sums[:, 0, :], axis=0) / float(N)
    var = jnp.sum(sums[:, 1, :], axis=0) / float(N) - mean * mean
    var = jnp.maximum(var, 0.0)
    o = (y_ref[...] - mean) * lax.rsqrt(var + 1e-5) * gamma_ref[...] \
        + beta_ref[...]
    o = jnp.where(o > 0.0, o, jnp.exp(jnp.minimum(o, 0.0)) - 1.0)
    out_ref[...] = o + x_ref[...]


def kernel(x, edge_index, edge_attr, W, att_src, att_dst, W_edge, att_edge,
           bias, gamma, beta):
    blkdiag = jnp.kron(jnp.eye(H, dtype=jnp.float32),
                       jnp.ones((C, 1), dtype=jnp.float32))
    as8 = blkdiag * att_src.reshape(H * C)[:, None]
    ad8 = blkdiag * att_dst.reshape(H * C)[:, None]
    pad8 = jnp.zeros((D, 8), jnp.float32)
    as16 = jnp.concatenate([as8, pad8], axis=1)
    ad16 = jnp.concatenate([ad8, pad8], axis=1)
    we8 = jnp.sum(W_edge.reshape(ED, H, C) * att_edge.reshape(1, H, C),
                  axis=-1)
    we16 = jnp.concatenate([we8, jnp.zeros((ED, 8), jnp.float32)], axis=1)
    web = jnp.kron(jnp.eye(8, dtype=jnp.float32), we16)
    s8 = jnp.kron(jnp.eye(H, dtype=jnp.float32),
                  jnp.ones((1, C), dtype=jnp.float32))
    bias2 = bias.reshape(1, D)
    gamma2 = gamma.reshape(1, D)
    beta2 = beta.reshape(1, D)

    nb = 10
    bn_rows = N // nb
    xp, asrc, adst = pl.pallas_call(
        _prep_body,
        grid=(nb,),
        in_specs=[
            pl.BlockSpec((bn_rows, D), lambda i: (i, 0)),
            pl.BlockSpec((D, D), lambda i: (0, 0)),
            pl.BlockSpec((D, 16), lambda i: (0, 0)),
            pl.BlockSpec((D, 16), lambda i: (0, 0)),
        ],
        out_specs=[
            pl.BlockSpec((bn_rows, D), lambda i: (i, 0)),
            pl.BlockSpec((bn_rows, 16), lambda i: (i, 0)),
            pl.BlockSpec((bn_rows, 16), lambda i: (i, 0)),
        ],
        out_shape=[
            jax.ShapeDtypeStruct((N, D), jnp.float32),
            jax.ShapeDtypeStruct((N, 16), jnp.float32),
            jax.ShapeDtypeStruct((N, 16), jnp.float32),
        ],
    )(x, W, as16, ad16)

    eb = 10
    eb_rows = E // 8 // eb
    ae = pl.pallas_call(
        _edge_body,
        grid=(eb,),
        in_specs=[
            pl.BlockSpec((eb_rows, 8 * ED), lambda i: (i, 0)),
            pl.BlockSpec((8 * ED, D), lambda i: (0, 0)),
        ],
        out_specs=pl.BlockSpec((eb_rows, D), lambda i: (i, 0)),
        out_shape=jax.ShapeDtypeStruct((E // 8, D), jnp.float32),
    )(edge_attr.reshape(E // 8, 8 * ED), web)

    mesh = plsc.VectorSubcoreMesh(core_axis_name="c", subcore_axis_name="s")
    pm, pd = pl.kernel(
        _sc_body,
        out_type=(
            jax.ShapeDtypeStruct((NC, P, D), jnp.float32),
            jax.ShapeDtypeStruct((NC, P, 16), jnp.float32),
        ),
        mesh=mesh,
        compiler_params=pltpu.CompilerParams(needs_layout_passes=False,
                                             use_tc_tiling_on_sc=False),
        scratch_types=(
            [
                pltpu.VMEM((NI, K), jnp.int32),
                pltpu.VMEM((NI, K), jnp.int32),
                pltpu.VMEM((NB, K, 16), jnp.float32),
                pltpu.VMEM((NB, K, 16), jnp.float32),
                pltpu.VMEM((NB, K // 8, D), jnp.float32),
                pltpu.VMEM((NB, K, D), jnp.float32),
                pltpu.VMEM((NB, K, 16), jnp.float32),
                pltpu.VMEM_SHARED((P, D), jnp.float32),
                pltpu.VMEM_SHARED((P, 16), jnp.float32),
            ]
            + [pltpu.SemaphoreType.DMA] * (2 * NB + NI)
        ),
    )(edge_index[0], edge_index[1], asrc, adst, ae, xp)

    y, sums = pl.pallas_call(
        _bn_stats_body,
        grid=(nb,),
        in_specs=[
            pl.BlockSpec((NC, bn_rows, D), lambda i: (0, i, 0)),
            pl.BlockSpec((NC, bn_rows, 16), lambda i: (0, i, 0)),
            pl.BlockSpec((8, D), lambda i: (0, 0)),
            pl.BlockSpec((1, D), lambda i: (0, 0)),
        ],
        out_specs=[
            pl.BlockSpec((bn_rows, D), lambda i: (i, 0)),
            pl.BlockSpec((1, 2, D), lambda i: (i, 0, 0)),
        ],
        out_shape=[
            jax.ShapeDtypeStruct((N, D), jnp.float32),
            jax.ShapeDtypeStruct((nb, 2, D), jnp.float32),
        ],
    )(pm, pd, s8, bias2)

    out = pl.pallas_call(
        _bn_apply_body,
        grid=(nb,),
        in_specs=[
            pl.BlockSpec((bn_rows, D), lambda i: (i, 0)),
            pl.BlockSpec((bn_rows, D), lambda i: (i, 0)),
            pl.BlockSpec((nb, 2, D), lambda i: (0, 0, 0)),
            pl.BlockSpec((1, D), lambda i: (0, 0)),
            pl.BlockSpec((1, D), lambda i: (0, 0)),
        ],
        out_specs=pl.BlockSpec((bn_rows, D), lambda i: (i, 0)),
        out_shape=jax.ShapeDtypeStruct((N, D), jnp.float32),
    )(y, x, sums, gamma2, beta2)
    return out
